# Initial kernel scaffold; baseline (speedup 1.0000x reference)
import jax
import jax.numpy as jnp
from jax.experimental import pallas as pl


def kernel(ploc, plabel, gloc, glabel, dboxes):
    raise NotImplementedError("write your pallas kernel here")


# speedup vs baseline: 1.5170x; 1.5170x over previous
import jax
import jax.numpy as jnp
from jax.experimental import pallas as pl

_B, _C, _N = 128, 81, 8732


def _smooth_l1_sum(diff):
    a = jnp.abs(diff)
    return jnp.sum(jnp.where(a < 1.0, 0.5 * a * a, a - 0.5), axis=0, keepdims=True)


def _ce_loc_kernel(plabel_ref, ploc_ref, gloc_ref, glabel_ref, dboxes_ref,
                   closs_ref, locv_ref):
    x = plabel_ref[0]
    lab = glabel_ref[0]

    m = jnp.max(x, axis=0, keepdims=True)
    s = jnp.sum(jnp.exp(x - m), axis=0, keepdims=True)
    lse = jnp.log(s) + m
    cls = jax.lax.broadcasted_iota(jnp.int32, x.shape, 0)
    xg = jnp.sum(jnp.where(cls == lab, x, 0.0), axis=0, keepdims=True)
    closs_ref[0] = lse - xg

    p = ploc_ref[0]
    g = gloc_ref[0]
    d = dboxes_ref[0]
    gxy = (g[:2] - d[:2]) / d[2:]
    gwh = jnp.log(g[2:] / d[2:])
    lv = _smooth_l1_sum(p[:2] - gxy) + _smooth_l1_sum(p[2:] - gwh)
    locv_ref[0] = lv * (lab > 0).astype(jnp.float32)


_RB = 32


def _mining_kernel(closs_ref, locv_ref, glabel_ref, out_ref):
    closs = closs_ref[:, 0, :]
    lab = glabel_ref[:, 0, :]
    mask = lab > 0
    maskf = mask.astype(jnp.float32)
    npos = jnp.sum(mask.astype(jnp.int32), axis=1, keepdims=True)
    k = jnp.minimum(3 * npos, _N)

    con = jnp.where(mask, 0.0, closs)
    bits = jax.lax.bitcast_convert_type(con, jnp.int32)

    def vbody(i, t):
        cand = t | (jnp.int32(1) << (30 - i))
        c = jnp.sum((bits >= cand).astype(jnp.int32), axis=1, keepdims=True)
        return jnp.where(c >= k, cand, t)

    t = jax.lax.fori_loop(0, 31, vbody, jnp.zeros((_RB, 1), jnp.int32))

    gt = bits > t
    cnt_gt = jnp.sum(gt.astype(jnp.int32), axis=1, keepdims=True)
    sum_gt = jnp.sum(jnp.where(gt, closs, 0.0), axis=1, keepdims=True)
    need = k - cnt_gt

    eq = bits == t
    iota = jax.lax.broadcasted_iota(jnp.int32, (_RB, _N), 1)

    def tbody(i, r):
        cand = r | (jnp.int32(1) << (13 - i))
        c = jnp.sum((eq & (iota < cand)).astype(jnp.int32), axis=1,
                    keepdims=True)
        return jnp.where(c < need, cand, r)

    r = jax.lax.fori_loop(0, 14, tbody, jnp.zeros((_RB, 1), jnp.int32))
    tie_sel = eq & (iota <= r) & (need > 0)
    tie_sum = jnp.sum(jnp.where(tie_sel, closs, 0.0), axis=1, keepdims=True)

    con_loss = jnp.sum(closs * maskf, axis=1, keepdims=True) + sum_gt + tie_sum
    loc_loss = jnp.sum(locv_ref[:, 0, :], axis=1, keepdims=True)
    total = loc_loss + con_loss

    nposf = npos.astype(jnp.float32)
    per = total * (npos > 0).astype(jnp.float32) / jnp.maximum(nposf, 1e-6)

    @pl.when(pl.program_id(0) == 0)
    def _init():
        out_ref[...] = jnp.zeros_like(out_ref)

    out_ref[...] += jnp.sum(per, keepdims=True) / _B


def kernel(ploc, plabel, gloc, glabel, dboxes):
    glabel3 = glabel.astype(jnp.int32).reshape(_B, 1, _N)

    closs, locv = pl.pallas_call(
        _ce_loc_kernel,
        grid=(_B,),
        in_specs=[
            pl.BlockSpec((1, _C, _N), lambda b: (b, 0, 0)),
            pl.BlockSpec((1, 4, _N), lambda b: (b, 0, 0)),
            pl.BlockSpec((1, 4, _N), lambda b: (b, 0, 0)),
            pl.BlockSpec((1, 1, _N), lambda b: (b, 0, 0)),
            pl.BlockSpec((1, 4, _N), lambda b: (0, 0, 0)),
        ],
        out_specs=[
            pl.BlockSpec((1, 1, _N), lambda b: (b, 0, 0)),
            pl.BlockSpec((1, 1, _N), lambda b: (b, 0, 0)),
        ],
        out_shape=[
            jax.ShapeDtypeStruct((_B, 1, _N), jnp.float32),
            jax.ShapeDtypeStruct((_B, 1, _N), jnp.float32),
        ],
    )(plabel, ploc, gloc, glabel3, dboxes)

    out = pl.pallas_call(
        _mining_kernel,
        grid=(_B // _RB,),
        in_specs=[
            pl.BlockSpec((_RB, 1, _N), lambda i: (i, 0, 0)),
            pl.BlockSpec((_RB, 1, _N), lambda i: (i, 0, 0)),
            pl.BlockSpec((_RB, 1, _N), lambda i: (i, 0, 0)),
        ],
        out_specs=pl.BlockSpec((1, 1), lambda i: (0, 0)),
        out_shape=jax.ShapeDtypeStruct((1, 1), jnp.float32),
    )(closs, locv, glabel3)
    return out[0, 0]
